# baseline (speedup 1.0000x reference)
import dataclasses
import functools

import jax
import jax.numpy as jnp
from jax import lax
from jax.experimental import pallas as pl
from jax.experimental.pallas import tpu as pltpu
from jax.experimental.pallas import tpu_sc as plsc

B = 16384
K = 128
N = 1024
UI = 1.0
IJ = 1.0

NC, NS, L = 1, 16, 16
NW = NC * NS
BPW = B // NW
CHUNK = 128
NCHUNK = BPW // CHUNK
TCHUNK = B // CHUNK


def _tables_body(gUI, gIU, gIJ, gJI, gIT, gTI, bU, bI, bT, alpha,
                 sU, sI, sJ, sT, sR,
                 p1, p2, p3, aux, s1, s2, s3, sx, sem):
    dn = (((1,), (1,)), ((), ()))
    a = alpha[0]
    s1[...] = jnp.reshape(
        lax.dot_general(gUI[...], gIU[...], dn,
                        preferred_element_type=jnp.float32)
        + UI * jnp.reshape(bU[...], (N, 1)) + a, (N * N,))
    c1 = pltpu.make_async_copy(s1, p1, sem)
    c1.start()
    s2[...] = jnp.reshape(
        IJ * lax.dot_general(gIJ[...], gJI[...], dn,
                             preferred_element_type=jnp.float32)
        + jnp.reshape(bI[...], (N, 1)), (N * N,))
    c2 = pltpu.make_async_copy(s2, p2, sem)
    c2.start()
    s3[...] = jnp.reshape(
        lax.dot_general(gTI[...], gIT[...], dn,
                        preferred_element_type=jnp.float32)
        + jnp.reshape(bT[...], (N, 1)), (N * N,))
    c3 = pltpu.make_async_copy(s3, p3, sem)
    c3.start()
    u = sU[...]
    i = sI[...]
    j = sJ[...]
    t = sT[...]
    sx[0] = jnp.reshape(u * N + i, (TCHUNK, CHUNK))
    sx[1] = jnp.reshape(i * N + j, (TCHUNK, CHUNK))
    sx[2] = jnp.reshape(t * N + i, (TCHUNK, CHUNK))
    sx[3] = jnp.reshape(lax.bitcast_convert_type(sR[...], jnp.int32),
                        (TCHUNK, CHUNK))
    c4 = pltpu.make_async_copy(sx, aux, sem)
    c4.start()
    c1.wait()
    c2.wait()
    c3.wait()
    c4.wait()


_tables = pl.pallas_call(
    _tables_body,
    in_specs=[pl.BlockSpec(memory_space=pltpu.VMEM)] * 9
    + [pl.BlockSpec(memory_space=pltpu.SMEM)]
    + [pl.BlockSpec(memory_space=pltpu.VMEM)] * 5,
    out_specs=[pl.BlockSpec(memory_space=pltpu.MemorySpace.HBM)] * 4,
    out_shape=[jax.ShapeDtypeStruct((N * N,), jnp.float32)] * 3
    + [jax.ShapeDtypeStruct((4, TCHUNK, CHUNK), jnp.int32)],
    scratch_shapes=[pltpu.VMEM((N * N,), jnp.float32)] * 3
    + [pltpu.VMEM((4, TCHUNK, CHUNK), jnp.int32), pltpu.SemaphoreType.DMA],
)


_sc_mesh = plsc.VectorSubcoreMesh(core_axis_name="c", subcore_axis_name="s",
                                  num_cores=NC)

_sc_params = pltpu.CompilerParams()
if "needs_layout_passes" in pltpu.CompilerParams.__dataclass_fields__:
    _sc_params = dataclasses.replace(_sc_params, needs_layout_passes=False)


@functools.partial(
    pl.kernel,
    out_type=jax.ShapeDtypeStruct((NW, L), jnp.float32),
    mesh=_sc_mesh,
    scratch_types=[
        pltpu.VMEM((4, NCHUNK, CHUNK), jnp.int32),
        pltpu.VMEM((NCHUNK, CHUNK), jnp.float32),
        pltpu.VMEM((NCHUNK, CHUNK), jnp.float32),
        pltpu.VMEM((NCHUNK, CHUNK), jnp.float32),
        pltpu.VMEM((L,), jnp.float32),
        pltpu.SemaphoreType.DMA,
    ],
    compiler_params=_sc_params,
)
def _sc_loss(p1_hbm, p2_hbm, p3_hbm, aux_hbm,
             out_hbm, a_v, g1, g2, g3, acc, sem):
    wid = lax.axis_index("s") * NC + lax.axis_index("c")
    cbase = wid * NCHUNK
    pltpu.async_copy(
        aux_hbm.at[:, pl.ds(cbase, NCHUNK), :], a_v, sem).wait()

    copies = []
    for c in range(NCHUNK):
        copies.append(pltpu.async_copy(p1_hbm.at[a_v.at[0, c]],
                                       g1.at[c], sem))
        copies.append(pltpu.async_copy(p2_hbm.at[a_v.at[1, c]],
                                       g2.at[c], sem))
        copies.append(pltpu.async_copy(p3_hbm.at[a_v.at[2, c]],
                                       g3.at[c], sem))
    for cp in copies:
        cp.wait()

    total = jnp.zeros((L,), jnp.float32)
    for c in range(NCHUNK):
        for o in range(CHUNK // L):
            sl = pl.ds(o * L, L)
            pred = g1[c, sl] + g2[c, sl] + g3[c, sl]
            d = pred - plsc.bitcast(a_v[3, c, sl], jnp.float32)
            total = total + d * d
    acc[...] = total
    pltpu.sync_copy(acc, out_hbm.at[wid])


def kernel(sampleT, sampleU, sampleI, sampleJ, sampleR, alpha, betaU, betaI,
           betaT, gammaUI, gammaIU, gammaIJ, gammaJI, gammaIT, gammaTI):
    p1, p2, p3, aux = _tables(
        gammaUI, gammaIU, gammaIJ, gammaJI, gammaIT, gammaTI,
        betaU, betaI, betaT, alpha.reshape(1),
        sampleU, sampleI, sampleJ, sampleT, sampleR,
    )
    partials = _sc_loss(p1, p2, p3, aux)
    return 0.5 * jnp.sum(partials) / B


# speedup vs baseline: 1.0339x; 1.0339x over previous
import dataclasses
import functools

import jax
import jax.numpy as jnp
from jax import lax
from jax.experimental import pallas as pl
from jax.experimental.pallas import tpu as pltpu
from jax.experimental.pallas import tpu_sc as plsc

B = 16384
K = 128
N = 1024
UI = 1.0
IJ = 1.0

NC, NS, L = 2, 16, 16
NW = NC * NS
BPW = B // NW
CHUNK = 128
NCHUNK = BPW // CHUNK
TCHUNK = B // CHUNK


def _tables_body(gUI, gIU, gIJ, gJI, gIT, gTI, bU, bI, bT, alpha,
                 sU, sI, sJ, sT, sR,
                 p1, p2, p3, aux, s1, s2, s3, sx, sem):
    dn = (((1,), (1,)), ((), ()))
    a = alpha[0]
    H = N // 2
    NN2 = H * N
    copies = []
    specs = [
        (gUI, gIU, bU, a, UI, s1, p1),
        (gIJ, gJI, bI, 0.0, IJ, s2, p2),
        (gTI, gIT, bT, 0.0, 1.0, s3, p3),
    ]
    for lhs, rhs, bias, extra, scale, sref, pref in specs:
        for h in range(2):
            rows = pl.ds(h * H, H)
            res = (scale * lax.dot_general(
                       lhs[rows, :], rhs[...], dn,
                       preferred_element_type=jnp.float32)
                   + jnp.reshape(bias[rows], (H, 1)) + extra)
            half = pl.ds(h * NN2, NN2)
            sref[half] = jnp.reshape(res, (NN2,))
            cc = pltpu.make_async_copy(sref.at[half], pref.at[half], sem)
            cc.start()
            copies.append(cc)
    u = sU[...]
    i = sI[...]
    j = sJ[...]
    t = sT[...]
    sx[0] = jnp.reshape(u * N + i, (TCHUNK, CHUNK))
    sx[1] = jnp.reshape(i * N + j, (TCHUNK, CHUNK))
    sx[2] = jnp.reshape(t * N + i, (TCHUNK, CHUNK))
    sx[3] = jnp.reshape(lax.bitcast_convert_type(sR[...], jnp.int32),
                        (TCHUNK, CHUNK))
    c4 = pltpu.make_async_copy(sx, aux, sem)
    c4.start()
    copies.append(c4)
    for cc in copies:
        cc.wait()


_tables = pl.pallas_call(
    _tables_body,
    in_specs=[pl.BlockSpec(memory_space=pltpu.VMEM)] * 9
    + [pl.BlockSpec(memory_space=pltpu.SMEM)]
    + [pl.BlockSpec(memory_space=pltpu.VMEM)] * 5,
    out_specs=[pl.BlockSpec(memory_space=pltpu.MemorySpace.HBM)] * 4,
    out_shape=[jax.ShapeDtypeStruct((N * N,), jnp.float32)] * 3
    + [jax.ShapeDtypeStruct((4, TCHUNK, CHUNK), jnp.int32)],
    scratch_shapes=[pltpu.VMEM((N * N,), jnp.float32)] * 3
    + [pltpu.VMEM((4, TCHUNK, CHUNK), jnp.int32), pltpu.SemaphoreType.DMA],
)


_sc_mesh = plsc.VectorSubcoreMesh(core_axis_name="c", subcore_axis_name="s")

_sc_params = pltpu.CompilerParams()
if "needs_layout_passes" in pltpu.CompilerParams.__dataclass_fields__:
    _sc_params = dataclasses.replace(_sc_params, needs_layout_passes=False)


@functools.partial(
    pl.kernel,
    out_type=jax.ShapeDtypeStruct((NW, L), jnp.float32),
    mesh=_sc_mesh,
    scratch_types=[
        pltpu.VMEM((4, NCHUNK, CHUNK), jnp.int32),
        pltpu.VMEM((NCHUNK, CHUNK), jnp.float32),
        pltpu.VMEM((NCHUNK, CHUNK), jnp.float32),
        pltpu.VMEM((NCHUNK, CHUNK), jnp.float32),
        pltpu.VMEM((L,), jnp.float32),
        pltpu.SemaphoreType.DMA,
    ],
    compiler_params=_sc_params,
)
def _sc_loss(p1_hbm, p2_hbm, p3_hbm, aux_hbm,
             out_hbm, a_v, g1, g2, g3, acc, sem):
    wid = lax.axis_index("s") * NC + lax.axis_index("c")
    cbase = wid * NCHUNK
    pltpu.async_copy(
        aux_hbm.at[:, pl.ds(cbase, NCHUNK), :], a_v, sem).wait()

    copies = []
    for c in range(NCHUNK):
        copies.append(pltpu.async_copy(p1_hbm.at[a_v.at[0, c]],
                                       g1.at[c], sem))
        copies.append(pltpu.async_copy(p2_hbm.at[a_v.at[1, c]],
                                       g2.at[c], sem))
        copies.append(pltpu.async_copy(p3_hbm.at[a_v.at[2, c]],
                                       g3.at[c], sem))
    for cp in copies:
        cp.wait()

    total = jnp.zeros((L,), jnp.float32)
    for c in range(NCHUNK):
        for o in range(CHUNK // L):
            sl = pl.ds(o * L, L)
            pred = g1[c, sl] + g2[c, sl] + g3[c, sl]
            d = pred - plsc.bitcast(a_v[3, c, sl], jnp.float32)
            total = total + d * d
    acc[...] = total
    pltpu.sync_copy(acc, out_hbm.at[wid])


def kernel(sampleT, sampleU, sampleI, sampleJ, sampleR, alpha, betaU, betaI,
           betaT, gammaUI, gammaIU, gammaIJ, gammaJI, gammaIT, gammaTI):
    p1, p2, p3, aux = _tables(
        gammaUI, gammaIU, gammaIJ, gammaJI, gammaIT, gammaTI,
        betaU, betaI, betaT, alpha.reshape(1),
        sampleU, sampleI, sampleJ, sampleT, sampleR,
    )
    partials = _sc_loss(p1, p2, p3, aux)
    return 0.5 * jnp.sum(partials) / B


# speedup vs baseline: 1.0471x; 1.0128x over previous
import dataclasses
import functools

import jax
import jax.numpy as jnp
from jax import lax
from jax.experimental import pallas as pl
from jax.experimental.pallas import tpu as pltpu
from jax.experimental.pallas import tpu_sc as plsc

B = 16384
K = 128
N = 1024
UI = 1.0
IJ = 1.0

NC, NS, L = 2, 16, 16
NW = NC * NS
BPW = B // NW
CHUNK = 128
NCHUNK = BPW // CHUNK
TCHUNK = B // CHUNK


def _tables_body(gUI, gIU, gIJ, gJI, gIT, gTI, bU, bI, bT, alpha,
                 sU, sI, sJ, sT, sR,
                 p1, p2, p3, aux, s1, s2, s3, sx, sem):
    dn = (((1,), (1,)), ((), ()))
    a = alpha[0]
    H = N // 4
    NN2 = H * N
    copies = []
    specs = [
        (gUI, gIU, bU, a, UI, s1, p1),
        (gIJ, gJI, bI, 0.0, IJ, s2, p2),
        (gTI, gIT, bT, 0.0, 1.0, s3, p3),
    ]
    for lhs, rhs, bias, extra, scale, sref, pref in specs:
        for h in range(4):
            rows = pl.ds(h * H, H)
            res = (scale * lax.dot_general(
                       lhs[rows, :], rhs[...], dn,
                       preferred_element_type=jnp.float32)
                   + jnp.reshape(bias[rows], (H, 1)) + extra)
            half = pl.ds(h * NN2, NN2)
            sref[half] = jnp.reshape(res, (NN2,))
            cc = pltpu.make_async_copy(sref.at[half], pref.at[half], sem)
            cc.start()
            copies.append(cc)
    u = sU[...]
    i = sI[...]
    j = sJ[...]
    t = sT[...]
    sx[0] = jnp.reshape(u * N + i, (TCHUNK, CHUNK))
    sx[1] = jnp.reshape(i * N + j, (TCHUNK, CHUNK))
    sx[2] = jnp.reshape(t * N + i, (TCHUNK, CHUNK))
    sx[3] = jnp.reshape(lax.bitcast_convert_type(sR[...], jnp.int32),
                        (TCHUNK, CHUNK))
    c4 = pltpu.make_async_copy(sx, aux, sem)
    c4.start()
    copies.append(c4)
    for cc in copies:
        cc.wait()


_tables = pl.pallas_call(
    _tables_body,
    in_specs=[pl.BlockSpec(memory_space=pltpu.VMEM)] * 9
    + [pl.BlockSpec(memory_space=pltpu.SMEM)]
    + [pl.BlockSpec(memory_space=pltpu.VMEM)] * 5,
    out_specs=[pl.BlockSpec(memory_space=pltpu.MemorySpace.HBM)] * 4,
    out_shape=[jax.ShapeDtypeStruct((N * N,), jnp.float32)] * 3
    + [jax.ShapeDtypeStruct((4, TCHUNK, CHUNK), jnp.int32)],
    scratch_shapes=[pltpu.VMEM((N * N,), jnp.float32)] * 3
    + [pltpu.VMEM((4, TCHUNK, CHUNK), jnp.int32), pltpu.SemaphoreType.DMA],
)


_sc_mesh = plsc.VectorSubcoreMesh(core_axis_name="c", subcore_axis_name="s")

_sc_params = pltpu.CompilerParams()
if "needs_layout_passes" in pltpu.CompilerParams.__dataclass_fields__:
    _sc_params = dataclasses.replace(_sc_params, needs_layout_passes=False)


@functools.partial(
    pl.kernel,
    out_type=jax.ShapeDtypeStruct((NW, L), jnp.float32),
    mesh=_sc_mesh,
    scratch_types=[
        pltpu.VMEM((4, NCHUNK, CHUNK), jnp.int32),
        pltpu.VMEM((NCHUNK, CHUNK), jnp.float32),
        pltpu.VMEM((NCHUNK, CHUNK), jnp.float32),
        pltpu.VMEM((NCHUNK, CHUNK), jnp.float32),
        pltpu.VMEM((L,), jnp.float32),
        pltpu.SemaphoreType.DMA,
    ],
    compiler_params=_sc_params,
)
def _sc_loss(p1_hbm, p2_hbm, p3_hbm, aux_hbm,
             out_hbm, a_v, g1, g2, g3, acc, sem):
    wid = lax.axis_index("s") * NC + lax.axis_index("c")
    cbase = wid * NCHUNK
    pltpu.async_copy(
        aux_hbm.at[:, pl.ds(cbase, NCHUNK), :], a_v, sem).wait()

    copies = []
    for c in range(NCHUNK):
        copies.append(pltpu.async_copy(p1_hbm.at[a_v.at[0, c]],
                                       g1.at[c], sem))
        copies.append(pltpu.async_copy(p2_hbm.at[a_v.at[1, c]],
                                       g2.at[c], sem))
        copies.append(pltpu.async_copy(p3_hbm.at[a_v.at[2, c]],
                                       g3.at[c], sem))
    for cp in copies:
        cp.wait()

    total = jnp.zeros((L,), jnp.float32)
    for c in range(NCHUNK):
        for o in range(CHUNK // L):
            sl = pl.ds(o * L, L)
            pred = g1[c, sl] + g2[c, sl] + g3[c, sl]
            d = pred - plsc.bitcast(a_v[3, c, sl], jnp.float32)
            total = total + d * d
    acc[...] = total
    pltpu.sync_copy(acc, out_hbm.at[wid])


def kernel(sampleT, sampleU, sampleI, sampleJ, sampleR, alpha, betaU, betaI,
           betaT, gammaUI, gammaIU, gammaIJ, gammaJI, gammaIT, gammaTI):
    p1, p2, p3, aux = _tables(
        gammaUI, gammaIU, gammaIJ, gammaJI, gammaIT, gammaTI,
        betaU, betaI, betaT, alpha.reshape(1),
        sampleU, sampleI, sampleJ, sampleT, sampleR,
    )
    partials = _sc_loss(p1, p2, p3, aux)
    return 0.5 * jnp.sum(partials) / B


# speedup vs baseline: 1.0566x; 1.0090x over previous
import dataclasses
import functools

import jax
import jax.numpy as jnp
from jax import lax
from jax.experimental import pallas as pl
from jax.experimental.pallas import tpu as pltpu
from jax.experimental.pallas import tpu_sc as plsc

B = 16384
K = 128
N = 1024
UI = 1.0
IJ = 1.0

NC, NS, L = 2, 16, 16
NW = NC * NS
BPW = B // NW
CHUNK = 128
NCHUNK = BPW // CHUNK
TCHUNK = B // CHUNK


def _tables_body(gUI, gIU, gIJ, gJI, gIT, gTI, bU, bI, bT, alpha,
                 sU, sI, sJ, sT, sR,
                 p1, p2, p3, aux, s1, s2, s3, sx, sem):
    dn = (((1,), (1,)), ((), ()))
    a = alpha[0]
    H = N // 8
    NN2 = H * N
    copies = []
    specs = [
        (gUI, gIU, bU, a, UI, s1, p1),
        (gIJ, gJI, bI, 0.0, IJ, s2, p2),
        (gTI, gIT, bT, 0.0, 1.0, s3, p3),
    ]
    for lhs, rhs, bias, extra, scale, sref, pref in specs:
        for h in range(8):
            rows = pl.ds(h * H, H)
            res = (scale * lax.dot_general(
                       lhs[rows, :], rhs[...], dn,
                       preferred_element_type=jnp.float32)
                   + jnp.reshape(bias[rows], (H, 1)) + extra)
            half = pl.ds(h * NN2, NN2)
            sref[half] = jnp.reshape(res, (NN2,))
            cc = pltpu.make_async_copy(sref.at[half], pref.at[half], sem)
            cc.start()
            copies.append(cc)
    u = sU[...]
    i = sI[...]
    j = sJ[...]
    t = sT[...]
    sx[0] = jnp.reshape(u * N + i, (TCHUNK, CHUNK))
    sx[1] = jnp.reshape(i * N + j, (TCHUNK, CHUNK))
    sx[2] = jnp.reshape(t * N + i, (TCHUNK, CHUNK))
    sx[3] = jnp.reshape(lax.bitcast_convert_type(sR[...], jnp.int32),
                        (TCHUNK, CHUNK))
    c4 = pltpu.make_async_copy(sx, aux, sem)
    c4.start()
    copies.append(c4)
    for cc in copies:
        cc.wait()


_tables = pl.pallas_call(
    _tables_body,
    in_specs=[pl.BlockSpec(memory_space=pltpu.VMEM)] * 9
    + [pl.BlockSpec(memory_space=pltpu.SMEM)]
    + [pl.BlockSpec(memory_space=pltpu.VMEM)] * 5,
    out_specs=[pl.BlockSpec(memory_space=pltpu.MemorySpace.HBM)] * 4,
    out_shape=[jax.ShapeDtypeStruct((N * N,), jnp.float32)] * 3
    + [jax.ShapeDtypeStruct((4, TCHUNK, CHUNK), jnp.int32)],
    scratch_shapes=[pltpu.VMEM((N * N,), jnp.float32)] * 3
    + [pltpu.VMEM((4, TCHUNK, CHUNK), jnp.int32), pltpu.SemaphoreType.DMA],
)


_sc_mesh = plsc.VectorSubcoreMesh(core_axis_name="c", subcore_axis_name="s")

_sc_params = pltpu.CompilerParams()
if "needs_layout_passes" in pltpu.CompilerParams.__dataclass_fields__:
    _sc_params = dataclasses.replace(_sc_params, needs_layout_passes=False)


@functools.partial(
    pl.kernel,
    out_type=jax.ShapeDtypeStruct((NW, L), jnp.float32),
    mesh=_sc_mesh,
    scratch_types=[
        pltpu.VMEM((4, NCHUNK, CHUNK), jnp.int32),
        pltpu.VMEM((NCHUNK, CHUNK), jnp.float32),
        pltpu.VMEM((NCHUNK, CHUNK), jnp.float32),
        pltpu.VMEM((NCHUNK, CHUNK), jnp.float32),
        pltpu.VMEM((L,), jnp.float32),
        pltpu.SemaphoreType.DMA,
    ],
    compiler_params=_sc_params,
)
def _sc_loss(p1_hbm, p2_hbm, p3_hbm, aux_hbm,
             out_hbm, a_v, g1, g2, g3, acc, sem):
    wid = lax.axis_index("s") * NC + lax.axis_index("c")
    cbase = wid * NCHUNK
    pltpu.async_copy(
        aux_hbm.at[:, pl.ds(cbase, NCHUNK), :], a_v, sem).wait()

    copies = []
    for c in range(NCHUNK):
        copies.append(pltpu.async_copy(p1_hbm.at[a_v.at[0, c]],
                                       g1.at[c], sem))
        copies.append(pltpu.async_copy(p2_hbm.at[a_v.at[1, c]],
                                       g2.at[c], sem))
        copies.append(pltpu.async_copy(p3_hbm.at[a_v.at[2, c]],
                                       g3.at[c], sem))
    for cp in copies:
        cp.wait()

    total = jnp.zeros((L,), jnp.float32)
    for c in range(NCHUNK):
        for o in range(CHUNK // L):
            sl = pl.ds(o * L, L)
            pred = g1[c, sl] + g2[c, sl] + g3[c, sl]
            d = pred - plsc.bitcast(a_v[3, c, sl], jnp.float32)
            total = total + d * d
    acc[...] = total
    pltpu.sync_copy(acc, out_hbm.at[wid])


def kernel(sampleT, sampleU, sampleI, sampleJ, sampleR, alpha, betaU, betaI,
           betaT, gammaUI, gammaIU, gammaIJ, gammaJI, gammaIT, gammaTI):
    p1, p2, p3, aux = _tables(
        gammaUI, gammaIU, gammaIJ, gammaJI, gammaIT, gammaTI,
        betaU, betaI, betaT, alpha.reshape(1),
        sampleU, sampleI, sampleJ, sampleT, sampleR,
    )
    partials = _sc_loss(p1, p2, p3, aux)
    return 0.5 * jnp.sum(partials) / B
